# baseline (speedup 1.0000x reference)
import numpy as np
import jax
import jax.numpy as jnp
from jax.experimental import pallas as pl
from jax.experimental.pallas import tpu as pltpu

_ALPHA = 4
_CHUNK = 32


def kernel(frames):
    C, T, H, W = frames.shape
    Ts = T // _ALPHA
    idx = np.linspace(0, T - 1, Ts).astype(np.int32)
    nj = T // _CHUNK

    def body(in_ref, slow_ref, fast_ref, buf, sin, sout):
        cp = pltpu.make_async_copy(in_ref, buf, sin.at[0])
        cp.start()
        cp.wait()
        outs = [pltpu.make_async_copy(buf, fast_ref, sout)]
        for c in range(C):
            for p, g in enumerate(idx):
                outs.append(pltpu.make_async_copy(
                    buf.at[c, int(g)], slow_ref.at[c, p], sout))
        for o in outs:
            o.start()
        for o in outs:
            o.wait()

    slow, fast = pl.pallas_call(
        body,
        in_specs=[pl.BlockSpec(memory_space=pl.ANY)],
        out_specs=[
            pl.BlockSpec(memory_space=pl.ANY),
            pl.BlockSpec(memory_space=pl.ANY),
        ],
        out_shape=[
            jax.ShapeDtypeStruct((C, Ts, H, W), frames.dtype),
            jax.ShapeDtypeStruct((C, T, H, W), frames.dtype),
        ],
        scratch_shapes=[
            pltpu.VMEM((C, T, H, W), frames.dtype),
            pltpu.SemaphoreType.DMA((C * nj,)),
            pltpu.SemaphoreType.DMA,
        ],
    )(frames)

    return (slow, fast)


# speedup vs baseline: 1.0829x; 1.0829x over previous
import numpy as np
import jax
import jax.numpy as jnp
from jax.experimental import pallas as pl
from jax.experimental.pallas import tpu as pltpu

_ALPHA = 4
_RAMP = (2, 4, 8, 18)


def kernel(frames):
    C, T, H, W = frames.shape
    Ts = T // _ALPHA
    idx = np.linspace(0, T - 1, Ts).astype(np.int32)
    assert sum(_RAMP) == T
    bounds = np.concatenate([[0], np.cumsum(_RAMP)])

    def body(in_ref, slow_ref, fast_ref, buf, sin, sout):
        ins = []
        n = 0
        for c in range(C):
            for j in range(len(_RAMP)):
                lo, hi = int(bounds[j]), int(bounds[j + 1])
                sl = pl.ds(lo, hi - lo)
                cp = pltpu.make_async_copy(
                    in_ref.at[c, sl], buf.at[c, sl], sin.at[n])
                cp.start()
                ins.append((c, lo, hi, cp))
                n += 1
        outs = []
        for c, lo, hi, cp in ins:
            cp.wait()
            sl = pl.ds(lo, hi - lo)
            o = pltpu.make_async_copy(buf.at[c, sl], fast_ref.at[c, sl], sout)
            o.start()
            outs.append(o)
            for p, g in enumerate(idx):
                if lo <= g < hi:
                    o2 = pltpu.make_async_copy(
                        buf.at[c, int(g)], slow_ref.at[c, p], sout)
                    o2.start()
                    outs.append(o2)
        for o in outs:
            o.wait()

    slow, fast = pl.pallas_call(
        body,
        in_specs=[pl.BlockSpec(memory_space=pl.ANY)],
        out_specs=[
            pl.BlockSpec(memory_space=pl.ANY),
            pl.BlockSpec(memory_space=pl.ANY),
        ],
        out_shape=[
            jax.ShapeDtypeStruct((C, Ts, H, W), frames.dtype),
            jax.ShapeDtypeStruct((C, T, H, W), frames.dtype),
        ],
        scratch_shapes=[
            pltpu.VMEM((C, T, H, W), frames.dtype),
            pltpu.SemaphoreType.DMA((C * len(_RAMP),)),
            pltpu.SemaphoreType.DMA,
        ],
    )(frames)

    return (slow, fast)


# speedup vs baseline: 1.1230x; 1.0370x over previous
import numpy as np
import jax
import jax.numpy as jnp
from jax.experimental import pallas as pl
from jax.experimental.pallas import tpu as pltpu

_ALPHA = 4
_CHUNK = 32


def kernel(frames):
    C, T, H, W = frames.shape
    Ts = T // _ALPHA
    idx = np.linspace(0, T - 1, Ts).astype(np.int32)
    nj = T // _CHUNK

    def body(in_ref, slow_ref, fast_ref, buf, sin, sout):
        ins = []
        n = 0
        for c in range(C):
            for j in range(nj):
                sl = pl.ds(j * _CHUNK, _CHUNK)
                cp = pltpu.make_async_copy(
                    in_ref.at[c, sl], buf.at[c, sl], sin.at[n])
                cp.start()
                ins.append((c, j, cp))
                n += 1
        outs = []
        for c, j, cp in ins:
            cp.wait()
            sl = pl.ds(j * _CHUNK, _CHUNK)
            o = pltpu.make_async_copy(buf.at[c, sl], fast_ref.at[c, sl], sout)
            o.start()
            outs.append(o)
            lo, hi = j * _CHUNK, (j + 1) * _CHUNK
            for p, g in enumerate(idx):
                if lo <= g < hi:
                    o2 = pltpu.make_async_copy(
                        buf.at[c, int(g)], slow_ref.at[c, p], sout)
                    o2.start()
                    outs.append(o2)
        for o in outs:
            o.wait()

    slow, fast = pl.pallas_call(
        body,
        in_specs=[pl.BlockSpec(memory_space=pl.ANY)],
        out_specs=[
            pl.BlockSpec(memory_space=pl.ANY),
            pl.BlockSpec(memory_space=pl.ANY),
        ],
        out_shape=[
            jax.ShapeDtypeStruct((C, Ts, H, W), frames.dtype),
            jax.ShapeDtypeStruct((C, T, H, W), frames.dtype),
        ],
        scratch_shapes=[
            pltpu.VMEM((C, T, H, W), frames.dtype),
            pltpu.SemaphoreType.DMA((C * nj,)),
            pltpu.SemaphoreType.DMA,
        ],
    )(frames)

    return (slow, fast)
